# baseline (speedup 1.0000x reference)
import functools

import jax
import jax.numpy as jnp
from jax import lax
from jax.experimental import pallas as pl
from jax.experimental.pallas import tpu as pltpu
from jax.experimental.pallas import tpu_sc as plsc

_N = 16384
_C = 1000

_INFO = plsc.get_sparse_core_info()
_NC = _INFO.num_cores
_NS = _INFO.num_subcores
_NW = _NC * _NS
_RPW = _N // _NW
_G = _RPW // 16

_mesh = plsc.VectorSubcoreMesh(core_axis_name="c", subcore_axis_name="s")


@functools.partial(
    pl.kernel,
    mesh=_mesh,
    out_type=jax.ShapeDtypeStruct((_NW, 16), jnp.float32),
    scratch_types=[
        pltpu.VMEM((16, _C), jnp.float32),
        pltpu.VMEM((_RPW,), jnp.int32),
        pltpu.VMEM((16,), jnp.float32),
    ],
    compiler_params=pltpu.CompilerParams(needs_layout_passes=False),
)
def _sc_recall(true_hbm, logits_hbm, out_hbm, buf, tvec, cnt_v):
    wid = lax.axis_index("s") * _NC + lax.axis_index("c")
    base = wid * _RPW
    pltpu.sync_copy(true_hbm.at[pl.ds(base, _RPW)], tvec)

    lanes = lax.broadcasted_iota(jnp.int32, (16,), 0)

    ninf = jnp.full((16,), -jnp.inf, jnp.float32)
    zero = jnp.zeros((16,), jnp.int32)

    def group_body(g, acc):
        pltpu.sync_copy(logits_hbm.at[pl.ds(base + g * 16, 16)], buf)

        def col_body(i, carry):
            m0, m1, m2, m3, a0, a1, a2, a3 = carry
            for d in range(2):
                ic = 2 * i + d
                ibc = jnp.full((16,), ic, jnp.int32)
                v0 = plsc.load_gather(buf, [lanes, ibc])
                v1 = plsc.load_gather(buf, [lanes, ibc + 250])
                v2 = plsc.load_gather(buf, [lanes, ibc + 500])
                v3 = plsc.load_gather(buf, [lanes, ibc + 750])
                u0 = v0 > m0
                u1 = v1 > m1
                u2 = v2 > m2
                u3 = v3 > m3
                m0 = jnp.where(u0, v0, m0)
                m1 = jnp.where(u1, v1, m1)
                m2 = jnp.where(u2, v2, m2)
                m3 = jnp.where(u3, v3, m3)
                a0 = jnp.where(u0, ibc, a0)
                a1 = jnp.where(u1, ibc, a1)
                a2 = jnp.where(u2, ibc, a2)
                a3 = jnp.where(u3, ibc, a3)
            return m0, m1, m2, m3, a0, a1, a2, a3

        init = (ninf, ninf, ninf, ninf, zero, zero, zero, zero)
        m0, m1, m2, m3, a0, a1, a2, a3 = plsc.parallel_loop(
            0, 125, carry=init, unroll=4
        )(lambda i, carry: col_body(i, carry))

        m, am = m0, a0
        for s, (ms, rs) in enumerate(((m1, a1), (m2, a2), (m3, a3)), start=1):
            u = ms > m
            m = jnp.where(u, ms, m)
            am = jnp.where(u, rs + s * 250, am)

        t16 = tvec[pl.ds(g * 16, 16)]
        return acc + jnp.where(am == t16, 1, 0)

    acc = lax.fori_loop(0, _G, group_body, jnp.zeros((16,), jnp.int32))
    cnt_v[...] = acc.astype(jnp.float32)
    pltpu.sync_copy(cnt_v, out_hbm.at[wid])


def kernel(true, logits):
    parts = _sc_recall(true.astype(jnp.int32), logits)
    return jnp.sum(parts) * (1.0 / _N)


# speedup vs baseline: 2.6593x; 2.6593x over previous
import functools

import jax
import jax.numpy as jnp
from jax import lax
from jax.experimental import pallas as pl
from jax.experimental.pallas import tpu as pltpu
from jax.experimental.pallas import tpu_sc as plsc

_N = 16384
_C = 1000

_BTC = 2048
_R_TC = 12288
_R_SC = _N - _R_TC

_INFO = plsc.get_sparse_core_info()
_NC = _INFO.num_cores
_NS = _INFO.num_subcores
_NW = _NC * _NS
_RPW = _R_SC // _NW
_G = _RPW // 16

_mesh = plsc.VectorSubcoreMesh(core_axis_name="c", subcore_axis_name="s")


def _tc_body(t_ref, x_ref, o_ref):
    i = pl.program_id(0)

    @pl.when(i == 0)
    def _init():
        o_ref[...] = jnp.zeros((1, 1), jnp.float32)

    x = x_ref[...]
    m = jnp.max(x, axis=1, keepdims=True)
    col = lax.broadcasted_iota(jnp.int32, (_BTC, _C), 1)
    first = jnp.min(jnp.where(x == m, col, _C), axis=1)
    t = t_ref[0, 0, :]
    cnt = jnp.sum((first == t).astype(jnp.float32)).reshape(1, 1)
    o_ref[...] = o_ref[...] + cnt


def _tc_count(true, logits):
    grid = _R_TC // _BTC
    t3 = true.reshape(_N // _BTC, 1, _BTC)
    out = pl.pallas_call(
        _tc_body,
        grid=(grid,),
        in_specs=[
            pl.BlockSpec((1, 1, _BTC), lambda i: (i, 0, 0)),
            pl.BlockSpec((_BTC, _C), lambda i: (i, 0)),
        ],
        out_specs=pl.BlockSpec((1, 1), lambda i: (0, 0)),
        out_shape=jax.ShapeDtypeStruct((1, 1), jnp.float32),
    )(t3, logits)
    return out[0, 0]


@functools.partial(
    pl.kernel,
    mesh=_mesh,
    out_type=jax.ShapeDtypeStruct((_NW, 16), jnp.float32),
    scratch_types=[
        pltpu.VMEM((16, _C), jnp.float32),
        pltpu.VMEM((_RPW,), jnp.int32),
        pltpu.VMEM((16,), jnp.float32),
    ],
    compiler_params=pltpu.CompilerParams(
        use_tc_tiling_on_sc=False, needs_layout_passes=False
    ),
)
def _sc_recall(true_hbm, logits_hbm, out_hbm, buf, tvec, cnt_v):
    wid = lax.axis_index("s") * _NC + lax.axis_index("c")
    base = wid * _RPW
    pltpu.sync_copy(true_hbm.at[pl.ds(base, _RPW)], tvec)

    lanes = lax.broadcasted_iota(jnp.int32, (16,), 0)
    ninf = jnp.full((16,), -jnp.inf, jnp.float32)
    zero = jnp.zeros((16,), jnp.int32)

    def group_body(g, acc):
        pltpu.sync_copy(logits_hbm.at[pl.ds(base + g * 16, 16)], buf)

        def col_body(i, carry):
            m0, m1, m2, m3, a0, a1, a2, a3 = carry
            for d in range(2):
                ic = 2 * i + d
                ibc = jnp.full((16,), ic, jnp.int32)
                v0 = plsc.load_gather(buf, [lanes, ibc])
                v1 = plsc.load_gather(buf, [lanes, ibc + 250])
                v2 = plsc.load_gather(buf, [lanes, ibc + 500])
                v3 = plsc.load_gather(buf, [lanes, ibc + 750])
                u0 = v0 > m0
                u1 = v1 > m1
                u2 = v2 > m2
                u3 = v3 > m3
                m0 = jnp.where(u0, v0, m0)
                m1 = jnp.where(u1, v1, m1)
                m2 = jnp.where(u2, v2, m2)
                m3 = jnp.where(u3, v3, m3)
                a0 = jnp.where(u0, ibc, a0)
                a1 = jnp.where(u1, ibc, a1)
                a2 = jnp.where(u2, ibc, a2)
                a3 = jnp.where(u3, ibc, a3)
            return m0, m1, m2, m3, a0, a1, a2, a3

        init = (ninf, ninf, ninf, ninf, zero, zero, zero, zero)
        m0, m1, m2, m3, a0, a1, a2, a3 = plsc.parallel_loop(
            0, 125, carry=init, unroll=4
        )(lambda i, carry: col_body(i, carry))

        m, am = m0, a0
        for s, (ms, rs) in enumerate(((m1, a1), (m2, a2), (m3, a3)), start=1):
            u = ms > m
            m = jnp.where(u, ms, m)
            am = jnp.where(u, rs + s * 250, am)

        t16 = tvec[pl.ds(g * 16, 16)]
        return acc + jnp.where(am == t16, 1, 0)

    acc = lax.fori_loop(0, _G, group_body, jnp.zeros((16,), jnp.int32))
    cnt_v[...] = acc.astype(jnp.float32)
    pltpu.sync_copy(cnt_v, out_hbm.at[wid])


def kernel(true, logits):
    true = true.astype(jnp.int32)
    tc_cnt = _tc_count(true, logits)
    sc_parts = _sc_recall(true[_R_TC:], logits[_R_TC:])
    return (tc_cnt + jnp.sum(sc_parts)) * (1.0 / _N)


# speedup vs baseline: 2.6629x; 1.0014x over previous
import functools

import jax
import jax.numpy as jnp
from jax import lax
from jax.experimental import pallas as pl
from jax.experimental.pallas import tpu as pltpu
from jax.experimental.pallas import tpu_sc as plsc

_N = 16384
_C = 1000

_BTC = 2048
_R_TC = 12288
_R_SC = _N - _R_TC

_INFO = plsc.get_sparse_core_info()
_NC = _INFO.num_cores
_NS = _INFO.num_subcores
_NW = _NC * _NS
_RPW = _R_SC // _NW
_G = _RPW // 16

_mesh = plsc.VectorSubcoreMesh(core_axis_name="c", subcore_axis_name="s")


def _tc_body(t_ref, x_ref, o_ref):
    i = pl.program_id(0)

    @pl.when(i == 0)
    def _init():
        o_ref[...] = jnp.zeros((1, 1), jnp.float32)

    x = x_ref[...]
    m = jnp.max(x, axis=1, keepdims=True)
    col = lax.broadcasted_iota(jnp.int32, (_BTC, _C), 1)
    first = jnp.min(jnp.where(x == m, col, _C), axis=1)
    t = t_ref[0, 0, :]
    cnt = jnp.sum((first == t).astype(jnp.float32)).reshape(1, 1)
    o_ref[...] = o_ref[...] + cnt


def _tc_count(true, logits):
    grid = _R_TC // _BTC
    t3 = true.reshape(_N // _BTC, 1, _BTC)
    out = pl.pallas_call(
        _tc_body,
        grid=(grid,),
        in_specs=[
            pl.BlockSpec((1, 1, _BTC), lambda i: (i, 0, 0)),
            pl.BlockSpec((_BTC, _C), lambda i: (i, 0)),
        ],
        out_specs=pl.BlockSpec((1, 1), lambda i: (0, 0)),
        out_shape=jax.ShapeDtypeStruct((1, 1), jnp.float32),
    )(t3, logits)
    return out[0, 0]


@functools.partial(
    pl.kernel,
    mesh=_mesh,
    out_type=jax.ShapeDtypeStruct((_NW, 16), jnp.float32),
    scratch_types=[
        pltpu.VMEM((16, _C), jnp.float32),
        pltpu.VMEM((_RPW,), jnp.int32),
        pltpu.VMEM((16,), jnp.float32),
    ],
    compiler_params=pltpu.CompilerParams(
        use_tc_tiling_on_sc=False, needs_layout_passes=False
    ),
)
def _sc_recall(true_hbm, logits_hbm, out_hbm, buf, tvec, cnt_v):
    wid = lax.axis_index("s") * _NC + lax.axis_index("c")
    base = wid * _RPW
    pltpu.sync_copy(true_hbm.at[pl.ds(base, _RPW)], tvec)

    lanes = lax.broadcasted_iota(jnp.int32, (16,), 0)
    ninf = jnp.full((16,), -jnp.inf, jnp.float32)
    zero = jnp.zeros((16,), jnp.int32)

    def group_body(g, acc):
        pltpu.sync_copy(logits_hbm.at[pl.ds(base + g * 16, 16)], buf)

        def col_body(i, carry):
            m0, m1, m2, m3, a0, a1, a2, a3 = carry
            for d in range(2):
                ic = 2 * i + d
                ibc = jnp.full((16,), ic, jnp.int32)
                v0 = plsc.load_gather(buf, [lanes, ibc])
                v1 = plsc.load_gather(buf, [lanes, ibc + 250])
                v2 = plsc.load_gather(buf, [lanes, ibc + 500])
                v3 = plsc.load_gather(buf, [lanes, ibc + 750])
                u0 = v0 > m0
                u1 = v1 > m1
                u2 = v2 > m2
                u3 = v3 > m3
                m0 = jnp.where(u0, v0, m0)
                m1 = jnp.where(u1, v1, m1)
                m2 = jnp.where(u2, v2, m2)
                m3 = jnp.where(u3, v3, m3)
                a0 = jnp.where(u0, ibc, a0)
                a1 = jnp.where(u1, ibc, a1)
                a2 = jnp.where(u2, ibc, a2)
                a3 = jnp.where(u3, ibc, a3)
            return m0, m1, m2, m3, a0, a1, a2, a3

        init = (ninf, ninf, ninf, ninf, zero, zero, zero, zero)
        m0, m1, m2, m3, a0, a1, a2, a3 = plsc.parallel_loop(
            0, 125, carry=init, unroll=4
        )(lambda i, carry: col_body(i, carry))

        m, am = m0, a0
        for s, (ms, rs) in enumerate(((m1, a1), (m2, a2), (m3, a3)), start=1):
            u = ms > m
            m = jnp.where(u, ms, m)
            am = jnp.where(u, rs + s * 250, am)

        t16 = tvec[pl.ds(g * 16, 16)]
        return acc + jnp.where(am == t16, 1, 0)

    acc = lax.fori_loop(0, _G, group_body, jnp.zeros((16,), jnp.int32))
    cnt_v[...] = acc.astype(jnp.float32)
    pltpu.sync_copy(cnt_v, out_hbm.at[wid])


def kernel(true, logits):
    true = true.astype(jnp.int32)
    sc_parts = _sc_recall(true[_R_TC:], logits[_R_TC:])
    tc_cnt = _tc_count(true, logits)
    return (tc_cnt + jnp.sum(sc_parts)) * (1.0 / _N)


# speedup vs baseline: 3.0278x; 1.1370x over previous
import functools

import jax
import jax.numpy as jnp
from jax import lax
from jax.experimental import pallas as pl
from jax.experimental.pallas import tpu as pltpu
from jax.experimental.pallas import tpu_sc as plsc

_N = 16384
_C = 1000

_BTC = 2048
_R_TC = 14336
_R_SC = _N - _R_TC

_INFO = plsc.get_sparse_core_info()
_NC = _INFO.num_cores
_NS = _INFO.num_subcores
_NW = _NC * _NS
_RPW = _R_SC // _NW
_G = _RPW // 16

_mesh = plsc.VectorSubcoreMesh(core_axis_name="c", subcore_axis_name="s")


def _tc_body(t_ref, x_ref, o_ref):
    i = pl.program_id(0)

    @pl.when(i == 0)
    def _init():
        o_ref[...] = jnp.zeros((1, 1), jnp.float32)

    x = x_ref[...]
    m = jnp.max(x, axis=1, keepdims=True)
    col = lax.broadcasted_iota(jnp.int32, (_BTC, _C), 1)
    first = jnp.min(jnp.where(x == m, col, _C), axis=1)
    t = t_ref[0, 0, :]
    cnt = jnp.sum((first == t).astype(jnp.float32)).reshape(1, 1)
    o_ref[...] = o_ref[...] + cnt


def _tc_count(true, logits):
    grid = _R_TC // _BTC
    t3 = true.reshape(_N // _BTC, 1, _BTC)
    out = pl.pallas_call(
        _tc_body,
        grid=(grid,),
        in_specs=[
            pl.BlockSpec((1, 1, _BTC), lambda i: (i, 0, 0)),
            pl.BlockSpec((_BTC, _C), lambda i: (i, 0)),
        ],
        out_specs=pl.BlockSpec((1, 1), lambda i: (0, 0)),
        out_shape=jax.ShapeDtypeStruct((1, 1), jnp.float32),
    )(t3, logits)
    return out[0, 0]


@functools.partial(
    pl.kernel,
    mesh=_mesh,
    out_type=jax.ShapeDtypeStruct((_NW, 16), jnp.float32),
    scratch_types=[
        pltpu.VMEM((16, _C), jnp.float32),
        pltpu.VMEM((_RPW,), jnp.int32),
        pltpu.VMEM((16,), jnp.float32),
    ],
    compiler_params=pltpu.CompilerParams(
        use_tc_tiling_on_sc=False, needs_layout_passes=False
    ),
)
def _sc_recall(true_hbm, logits_hbm, out_hbm, buf, tvec, cnt_v):
    wid = lax.axis_index("s") * _NC + lax.axis_index("c")
    base = wid * _RPW
    pltpu.sync_copy(true_hbm.at[pl.ds(base, _RPW)], tvec)

    lanes = lax.broadcasted_iota(jnp.int32, (16,), 0)
    ninf = jnp.full((16,), -jnp.inf, jnp.float32)
    zero = jnp.zeros((16,), jnp.int32)

    def group_body(g, acc):
        pltpu.sync_copy(logits_hbm.at[pl.ds(base + g * 16, 16)], buf)

        def col_body(i, carry):
            m0, m1, m2, m3, a0, a1, a2, a3 = carry
            for d in range(2):
                ic = 2 * i + d
                ibc = jnp.full((16,), ic, jnp.int32)
                v0 = plsc.load_gather(buf, [lanes, ibc])
                v1 = plsc.load_gather(buf, [lanes, ibc + 250])
                v2 = plsc.load_gather(buf, [lanes, ibc + 500])
                v3 = plsc.load_gather(buf, [lanes, ibc + 750])
                u0 = v0 > m0
                u1 = v1 > m1
                u2 = v2 > m2
                u3 = v3 > m3
                m0 = jnp.where(u0, v0, m0)
                m1 = jnp.where(u1, v1, m1)
                m2 = jnp.where(u2, v2, m2)
                m3 = jnp.where(u3, v3, m3)
                a0 = jnp.where(u0, ibc, a0)
                a1 = jnp.where(u1, ibc, a1)
                a2 = jnp.where(u2, ibc, a2)
                a3 = jnp.where(u3, ibc, a3)
            return m0, m1, m2, m3, a0, a1, a2, a3

        init = (ninf, ninf, ninf, ninf, zero, zero, zero, zero)
        m0, m1, m2, m3, a0, a1, a2, a3 = plsc.parallel_loop(
            0, 125, carry=init, unroll=4
        )(lambda i, carry: col_body(i, carry))

        m, am = m0, a0
        for s, (ms, rs) in enumerate(((m1, a1), (m2, a2), (m3, a3)), start=1):
            u = ms > m
            m = jnp.where(u, ms, m)
            am = jnp.where(u, rs + s * 250, am)

        t16 = tvec[pl.ds(g * 16, 16)]
        return acc + jnp.where(am == t16, 1, 0)

    acc = lax.fori_loop(0, _G, group_body, jnp.zeros((16,), jnp.int32))
    cnt_v[...] = acc.astype(jnp.float32)
    pltpu.sync_copy(cnt_v, out_hbm.at[wid])


def kernel(true, logits):
    true = true.astype(jnp.int32)
    sc_parts = _sc_recall(true[_R_TC:], logits[_R_TC:])
    tc_cnt = _tc_count(true, logits)
    return (tc_cnt + jnp.sum(sc_parts)) * (1.0 / _N)
